# baseline (speedup 1.0000x reference)
import functools

import jax
import jax.numpy as jnp
from jax import lax
from jax.experimental import pallas as pl
from jax.experimental.pallas import tpu as pltpu
from jax.experimental.pallas import tpu_sc as plsc

_NUM = 13
_CAT = 26
_V = 100000
_VP = 100096
_D = 32
_F = _NUM + _CAT

_NC = 2
_NS = 16
_NW = _NC * _NS
_NB = 128
_VCH = 128
_NCH = _V // _VCH


def _transpose_chunk(staged, slab, nv, unroll):
  k0 = lax.iota(jnp.int32, 16) // 8
  k1 = k0 + 2
  r = lax.iota(jnp.int32, 16) % 8

  def grp_body(g, _):
    for i in range(unroll):
      v = g * unroll + i
      vv = jnp.full((16,), 0, jnp.int32) + v
      lo = plsc.load_gather(staged, [k0, r, vv])
      hi = plsc.load_gather(staged, [k1, r, vv])
      slab[v, pl.ds(0, 16)] = lo
      slab[v, pl.ds(16, 16)] = hi
    return 0

  lax.fori_loop(0, nv // unroll, grp_body, 0)


def _relayout_body(tab4, out_hbm, staged, slab, staged_t, slab_t, sem):
  wid = lax.axis_index("s") * _NC + lax.axis_index("c")

  def feat_body(c, _):
    def chunk_body(j, _):
      ch = wid + j * _NW
      @pl.when(ch < _NCH)
      def _():
        v0 = ch * _VCH
        descs = []
        for k in range(4):
          descs.append(pltpu.async_copy(
              tab4.at[c, k, :, pl.ds(v0, _VCH)], staged.at[k], sem))
        for dd in descs:
          dd.wait()
        _transpose_chunk(staged, slab, _VCH, 8)
        pltpu.sync_copy(slab, out_hbm.at[c, pl.ds(v0, _VCH), :])
      return 0

    nj = (_NCH + _NW - 1) // _NW
    lax.fori_loop(0, nj, chunk_body, 0)

    @pl.when(wid == 0)
    def _():
      v0 = _NCH * _VCH
      descs = []
      for k in range(4):
        descs.append(pltpu.async_copy(
            tab4.at[c, k, :, pl.ds(v0, _V - v0)], staged_t.at[k], sem))
      for dd in descs:
        dd.wait()
      _transpose_chunk(staged_t, slab_t, _V - v0, 8)
      pltpu.sync_copy(slab_t, out_hbm.at[c, pl.ds(v0, _V - v0), :])
    return 0

  lax.fori_loop(0, _CAT, feat_body, 0)


def _sc_relayout(tab_t):
  tab4 = tab_t.reshape(_CAT, 4, 8, _V)
  mesh = plsc.VectorSubcoreMesh(core_axis_name="c", subcore_axis_name="s")
  return pl.kernel(
      _relayout_body,
      out_type=jax.ShapeDtypeStruct((_CAT, _VP, _D), jnp.float32),
      mesh=mesh,
      scratch_types=[
          pltpu.VMEM((4, 8, _VCH), jnp.float32),
          pltpu.VMEM((_VCH, _D), jnp.float32),
          pltpu.VMEM((4, 8, _V - _NCH * _VCH), jnp.float32),
          pltpu.VMEM((_V - _NCH * _VCH, _D), jnp.float32),
          pltpu.SemaphoreType.DMA,
      ],
      compiler_params=pltpu.CompilerParams(needs_layout_passes=False),
  )(tab4)


def _gather_body(x_hbm, tab_hbm, out_hbm, xb, idx_v, gbuf, outb, sem):
  wid = lax.axis_index("s") * _NC + lax.axis_index("c")
  b = x_hbm.shape[0]
  b_per_w = b // _NW
  nblk = b_per_w // _NB

  def blk_body(blk, _):
    base = wid * b_per_w + blk * _NB
    pltpu.sync_copy(x_hbm.at[pl.ds(base, _NB), :], xb)
    lanes = lax.iota(jnp.int32, 16)
    for c in range(_CAT):
      col = jnp.full((16,), _NUM + c, jnp.int32)
      for j in range(_NB // 16):
        v = plsc.load_gather(xb, [lanes + (j * 16), col])
        idx_v[c, pl.ds(j * 16, 16)] = v.astype(jnp.int32)
    descs = []
    for c in range(_CAT):
      descs.append(
          pltpu.async_copy(tab_hbm.at[c].at[idx_v.at[c]], gbuf.at[c], sem))
    for d in descs:
      d.wait()
    def row_body(r, _):
      a0 = gbuf[0, r, pl.ds(0, 16)]
      a1 = gbuf[0, r, pl.ds(16, 16)]
      for c in range(1, _CAT):
        a0 = a0 + gbuf[c, r, pl.ds(0, 16)]
        a1 = a1 + gbuf[c, r, pl.ds(16, 16)]
      outb[r, pl.ds(0, 16)] = a0
      outb[r, pl.ds(16, 16)] = a1
      return 0
    lax.fori_loop(0, _NB, row_body, 0)
    pltpu.sync_copy(outb, out_hbm.at[pl.ds(base, _NB)])
    return 0

  lax.fori_loop(0, nblk, blk_body, 0)


def _sc_gather_sum(x, tab_relaid):
  b = x.shape[0]
  mesh = plsc.VectorSubcoreMesh(core_axis_name="c", subcore_axis_name="s")
  return pl.kernel(
      _gather_body,
      out_type=jax.ShapeDtypeStruct((b, _D), jnp.float32),
      mesh=mesh,
      scratch_types=[
          pltpu.VMEM((_NB, _F), jnp.float32),
          pltpu.VMEM((_CAT, _NB), jnp.int32),
          pltpu.VMEM((_CAT, _NB, _D), jnp.float32),
          pltpu.VMEM((_NB, _D), jnp.float32),
          pltpu.SemaphoreType.DMA,
      ],
      compiler_params=pltpu.CompilerParams(
          use_tc_tiling_on_sc=False, needs_layout_passes=False),
  )(x, tab_relaid)


def _tc_body(xn_ref, w_ref, b_ref, cs_ref, o_ref):
  xn = xn_ref[...]
  w = w_ref[...]
  bias_sum = jnp.sum(b_ref[...], axis=0, keepdims=True)
  num = jnp.dot(xn, w, preferred_element_type=jnp.float32)
  o_ref[...] = (num + bias_sum + cs_ref[...]) * (1.0 / _F)


def _tc_finalize(x_num, num_weights, num_biases, cat_sum):
  b = x_num.shape[0]
  bt = 4096
  grid = b // bt
  return pl.pallas_call(
      _tc_body,
      grid=(grid,),
      in_specs=[
          pl.BlockSpec((bt, _NUM), lambda i: (i, 0)),
          pl.BlockSpec((_NUM, _D), lambda i: (0, 0)),
          pl.BlockSpec((_NUM, _D), lambda i: (0, 0)),
          pl.BlockSpec((bt, _D), lambda i: (i, 0)),
      ],
      out_specs=pl.BlockSpec((bt, _D), lambda i: (i, 0)),
      out_shape=jax.ShapeDtypeStruct((b, _D), jnp.float32),
  )(x_num, num_weights, num_biases, cat_sum)


@jax.jit
def kernel(x, tables, num_weights, num_biases):
  x_num = x[:, :_NUM]
  tab_t = jnp.transpose(tables, (0, 2, 1))
  tab_relaid = _sc_relayout(tab_t)
  cat_sum = _sc_gather_sum(x, tab_relaid)
  return _tc_finalize(x_num, num_weights, num_biases, cat_sum)


# speedup vs baseline: 1.5701x; 1.5701x over previous
import functools

import jax
import jax.numpy as jnp
from jax import lax
from jax.experimental import pallas as pl
from jax.experimental.pallas import tpu as pltpu
from jax.experimental.pallas import tpu_sc as plsc

_NUM = 13
_CAT = 26
_V = 100000
_VP = 100096
_D = 32
_F = _NUM + _CAT

_NC = 2
_NS = 16
_NW = _NC * _NS
_NB = 128
_VCH = 512
_NFC = _V // _VCH


def _transpose_rows(staged, slab, nv):
  k0 = lax.iota(jnp.int32, 16) // 8
  k1 = k0 + 2
  r = lax.iota(jnp.int32, 16) % 8

  def grp_body(g, _):
    for i in range(8):
      v = g * 8 + i
      vv = jnp.full((16,), 0, jnp.int32) + v
      q = 2 * g + i // 4
      col = (i % 4) * 32
      slab[q, pl.ds(col, 16)] = plsc.load_gather(staged, [k0, r, vv])
      slab[q, pl.ds(col + 16, 16)] = plsc.load_gather(staged, [k1, r, vv])
    return 0

  lax.fori_loop(0, nv // 8, grp_body, 0)


def _relayout_body(tab4, tab_tail, out_hbm, staged, slab, si0, si1, so0, so1):
  wid = lax.axis_index("s") * _NC + lax.axis_index("c")
  sem_in = (si0, si1)
  sem_out = (so0, so1)

  def in_copies(c, j, buf):
    ch = wid + j * _NW
    v0 = ch * _VCH
    return [
        pltpu.make_async_copy(
            tab4.at[c, k, :, pl.ds(v0, _VCH)],
            staged.at[buf, k], sem_in[buf])
        for k in range(4)
    ]

  def out_copy(c, j, buf):
    ch = wid + j * _NW
    r0 = ch * (_VCH * _D // 128)
    return pltpu.make_async_copy(
        slab.at[buf], out_hbm.at[c, pl.ds(r0, _VCH * _D // 128), :],
        sem_out[buf])

  def do_chunk(c, g, j, buf, nj):
    for d in in_copies(c, j, buf):
      d.wait()

    @pl.when(j >= 2)
    def _():
      out_copy(c, j - 2, buf).wait()

    _transpose_rows(staged.at[buf], slab.at[buf], _VCH)

    @pl.when(j + 2 < nj)
    def _():
      for d in in_copies(c, j + 2, buf):
        d.start()

    out_copy(c, j, buf).start()

  def feat_body(c, _):
    nj = (_NFC - wid + _NW - 1) // _NW

    @pl.when(nj > 0)
    def _():
      for d in in_copies(c, 0, 0):
        d.start()

      @pl.when(nj > 1)
      def _():
        for d in in_copies(c, 1, 1):
          d.start()

      def pair_body(g, _):
        do_chunk(c, g, 2 * g, 0, nj)

        @pl.when(2 * g + 1 < nj)
        def _():
          do_chunk(c, g, 2 * g + 1, 1, nj)
        return 0

      lax.fori_loop(0, (nj + 1) // 2, pair_body, 0)

      nlast = nj - 1
      out_copy(c, nlast - (nlast & 1), 0).wait()

      @pl.when(nj > 1)
      def _():
        out_copy(c, nlast - 1 + (nlast & 1), 1).wait()
    return 0

  lax.fori_loop(0, _CAT, feat_body, 0)

  for t in (wid, wid + _NW):
    is128 = t < _CAT
    is32 = jnp.logical_and(t >= _CAT, t < 2 * _CAT)
    c = t - jnp.where(t >= _CAT, _CAT, 0)

    @pl.when(is128)
    def _():
      v0 = _NFC * _VCH
      for k in range(4):
        pltpu.sync_copy(tab4.at[c, k, :, pl.ds(v0, 128)],
                        staged.at[0, k, :, pl.ds(0, 128)])
      _transpose_rows(staged.at[0], slab.at[0], 128)
      pltpu.sync_copy(slab.at[0, pl.ds(0, 32), :],
                      out_hbm.at[c, pl.ds(v0 * _D // 128, 32), :])

    @pl.when(is32)
    def _():
      r0 = (_NFC * _VCH + 128) * _D // 128
      pltpu.sync_copy(tab_tail.at[c], out_hbm.at[c, pl.ds(r0, 8), :])


def _sc_relayout(tab_t, tab_tail):
  tab4 = tab_t.reshape(_CAT, 4, 8, _V)
  mesh = plsc.VectorSubcoreMesh(core_axis_name="c", subcore_axis_name="s")
  return pl.kernel(
      _relayout_body,
      out_type=jax.ShapeDtypeStruct((_CAT, _VP * _D // 128, 128), jnp.float32),
      mesh=mesh,
      scratch_types=[
          pltpu.VMEM((2, 4, 8, _VCH), jnp.float32),
          pltpu.VMEM((2, _VCH * _D // 128, 128), jnp.float32),
          pltpu.SemaphoreType.DMA,
          pltpu.SemaphoreType.DMA,
          pltpu.SemaphoreType.DMA,
          pltpu.SemaphoreType.DMA,
      ],
      compiler_params=pltpu.CompilerParams(needs_layout_passes=False),
  )(tab4, tab_tail)


def _gather_body(x_hbm, tab_hbm, out_hbm, xb, idx_v, gbuf, outb, sem):
  wid = lax.axis_index("s") * _NC + lax.axis_index("c")
  b = x_hbm.shape[0]
  b_per_w = b // _NW
  nblk = b_per_w // _NB

  def blk_body(blk, _):
    base = wid * b_per_w + blk * _NB
    pltpu.sync_copy(x_hbm.at[pl.ds(base, _NB), :], xb)
    lanes = lax.iota(jnp.int32, 16)
    for c in range(_CAT):
      col = jnp.full((16,), _NUM + c, jnp.int32)
      for j in range(_NB // 16):
        v = plsc.load_gather(xb, [lanes + (j * 16), col])
        idx_v[c, pl.ds(j * 16, 16)] = v.astype(jnp.int32)
    descs = []
    for c in range(_CAT):
      descs.append(
          pltpu.async_copy(tab_hbm.at[c].at[idx_v.at[c]], gbuf.at[c], sem))
    for d in descs:
      d.wait()
    def row_body(r, _):
      a0 = gbuf[0, r, pl.ds(0, 16)]
      a1 = gbuf[0, r, pl.ds(16, 16)]
      for c in range(1, _CAT):
        a0 = a0 + gbuf[c, r, pl.ds(0, 16)]
        a1 = a1 + gbuf[c, r, pl.ds(16, 16)]
      outb[r, pl.ds(0, 16)] = a0
      outb[r, pl.ds(16, 16)] = a1
      return 0
    lax.fori_loop(0, _NB, row_body, 0)
    pltpu.sync_copy(outb, out_hbm.at[pl.ds(base, _NB)])
    return 0

  lax.fori_loop(0, nblk, blk_body, 0)


def _sc_gather_sum(x, tab_relaid):
  b = x.shape[0]
  mesh = plsc.VectorSubcoreMesh(core_axis_name="c", subcore_axis_name="s")
  return pl.kernel(
      _gather_body,
      out_type=jax.ShapeDtypeStruct((b, _D), jnp.float32),
      mesh=mesh,
      scratch_types=[
          pltpu.VMEM((_NB, _F), jnp.float32),
          pltpu.VMEM((_CAT, _NB), jnp.int32),
          pltpu.VMEM((_CAT, _NB, _D), jnp.float32),
          pltpu.VMEM((_NB, _D), jnp.float32),
          pltpu.SemaphoreType.DMA,
      ],
      compiler_params=pltpu.CompilerParams(
          use_tc_tiling_on_sc=False, needs_layout_passes=False),
  )(x, tab_relaid)


def _tc_body(xn_ref, w_ref, b_ref, cs_ref, o_ref):
  xn = xn_ref[...]
  w = w_ref[...]
  bias_sum = jnp.sum(b_ref[...], axis=0, keepdims=True)
  num = jnp.dot(xn, w, preferred_element_type=jnp.float32)
  o_ref[...] = (num + bias_sum + cs_ref[...]) * (1.0 / _F)


def _tc_finalize(x_num, num_weights, num_biases, cat_sum):
  b = x_num.shape[0]
  bt = 4096
  grid = b // bt
  return pl.pallas_call(
      _tc_body,
      grid=(grid,),
      in_specs=[
          pl.BlockSpec((bt, _NUM), lambda i: (i, 0)),
          pl.BlockSpec((_NUM, _D), lambda i: (0, 0)),
          pl.BlockSpec((_NUM, _D), lambda i: (0, 0)),
          pl.BlockSpec((bt, _D), lambda i: (i, 0)),
      ],
      out_specs=pl.BlockSpec((bt, _D), lambda i: (i, 0)),
      out_shape=jax.ShapeDtypeStruct((b, _D), jnp.float32),
  )(x_num, num_weights, num_biases, cat_sum)


@jax.jit
def kernel(x, tables, num_weights, num_biases):
  x_num = x[:, :_NUM]
  tab_t = jnp.transpose(tables, (0, 2, 1))
  tab_tail = lax.slice(tables, (0, _NFC * _VCH + 128, 0), (_CAT, _V, _D))
  tab_tail = tab_tail.reshape(_CAT, 8, 128)
  tab_relaid = _sc_relayout(tab_t, tab_tail)
  tab_relaid = tab_relaid.reshape(_CAT, _VP, _D)
  cat_sum = _sc_gather_sum(x, tab_relaid)
  return _tc_finalize(x_num, num_weights, num_biases, cat_sum)
